# baseline (speedup 1.0000x reference)
import functools

import jax
import jax.numpy as jnp
from jax import lax
from jax.experimental import pallas as pl
from jax.experimental.pallas import tpu as pltpu
from jax.experimental.pallas import tpu_sc as plsc

N_NODES = 10000
N_EDGES = 320000
D_FEAT = 128

NC = 2
NS = 16
NW = NC * NS

E_MAIN = 9984
N_TAIL = N_EDGES - NW * E_MAIN
TAIL_W = N_TAIL // 128
N_PAD = 10240
Z_W = N_PAD // NS

ROWS_BLK = 2048
N_BLKS = (N_NODES + ROWS_BLK - 1) // ROWS_BLK

_mesh = plsc.VectorSubcoreMesh(core_axis_name="c", subcore_axis_name="s")


@functools.partial(
    pl.kernel,
    out_type=jax.ShapeDtypeStruct((NC, N_PAD), jnp.float32),
    mesh=_mesh,
    scratch_types=[
        pltpu.VMEM((E_MAIN,), jnp.int32),
        pltpu.VMEM((128,), jnp.int32),
        pltpu.VMEM((E_MAIN,), jnp.float32),
        pltpu.VMEM((Z_W,), jnp.float32),
        pltpu.VMEM_SHARED((N_PAD,), jnp.float32),
    ],
)
def _degree_kernel(edge_hbm, ones_hbm, out_hbm, idx1_v, tail1_v,
                   ones_v, zeros_v, hist_sh):
    cid = lax.axis_index("c")
    sid = lax.axis_index("s")
    wid = sid * NC + cid

    def fillz(i, _):
        zeros_v[pl.ds(i * 16, 16)] = jnp.zeros((16,), jnp.float32)
        return 0

    lax.fori_loop(0, Z_W // 16, fillz, 0)

    pltpu.sync_copy(ones_hbm, ones_v)
    pltpu.sync_copy(edge_hbm.at[pl.ds(N_EDGES + wid * E_MAIN, E_MAIN)],
                    idx1_v)

    @pl.when(wid < TAIL_W)
    def _():
        pltpu.sync_copy(
            edge_hbm.at[pl.ds(N_EDGES + NW * E_MAIN + wid * 128, 128)],
            tail1_v)

    pltpu.sync_copy(zeros_v, hist_sh.at[pl.ds(sid * Z_W, Z_W)])
    plsc.subcore_barrier()

    pltpu.sync_copy(ones_v, hist_sh.at[idx1_v], add=True)

    @pl.when(wid < TAIL_W)
    def _():
        pltpu.sync_copy(ones_v.at[pl.ds(0, 128)],
                        hist_sh.at[tail1_v], add=True)

    plsc.subcore_barrier()

    @pl.when(sid == 0)
    def _():
        pltpu.sync_copy(hist_sh, out_hbm.at[cid])


def _mm_body(x_ref, w_ref, o_ref):
    o_ref[...] = jnp.dot(x_ref[...], w_ref[...],
                         preferred_element_type=jnp.float32)


def _scale_body(c_ref, xw_ref, o_ref):
    cnt = c_ref[0, :] + c_ref[1, :]
    o_ref[...] = xw_ref[...] * cnt[:, None]


def kernel(edge_index, x, W):
    edge_flat = edge_index.reshape(-1)
    ones_src = jnp.ones((E_MAIN,), jnp.float32)
    deg = _degree_kernel(edge_flat, ones_src)

    xw = pl.pallas_call(
        _mm_body,
        grid=(N_BLKS,),
        out_shape=jax.ShapeDtypeStruct((N_NODES, D_FEAT), jnp.float32),
        in_specs=[
            pl.BlockSpec((ROWS_BLK, D_FEAT), lambda i: (i, 0)),
            pl.BlockSpec((D_FEAT, D_FEAT), lambda i: (0, 0)),
        ],
        out_specs=pl.BlockSpec((ROWS_BLK, D_FEAT), lambda i: (i, 0)),
    )(x, W)

    out = pl.pallas_call(
        _scale_body,
        grid=(N_BLKS,),
        out_shape=jax.ShapeDtypeStruct((N_NODES, D_FEAT), jnp.float32),
        in_specs=[
            pl.BlockSpec((NC, ROWS_BLK), lambda i: (0, i)),
            pl.BlockSpec((ROWS_BLK, D_FEAT), lambda i: (i, 0)),
        ],
        out_specs=pl.BlockSpec((ROWS_BLK, D_FEAT), lambda i: (i, 0)),
    )(deg, xw)
    return out


# speedup vs baseline: 1.1568x; 1.1568x over previous
import functools

import jax
import jax.numpy as jnp
from jax import lax
from jax.experimental import pallas as pl
from jax.experimental.pallas import tpu as pltpu
from jax.experimental.pallas import tpu_sc as plsc

N_NODES = 10000
N_EDGES = 320000
D_FEAT = 128

NC = 2
NS = 16
NW = NC * NS

E_MAIN = 9984
N_TAIL = N_EDGES - NW * E_MAIN
TAIL_W = N_TAIL // 128
N_PAD = 10240
Z_W = N_PAD // NS
N_CH = 6
CH = E_MAIN // N_CH

ROWS_BLK = 2048
N_BLKS = (N_NODES + ROWS_BLK - 1) // ROWS_BLK

_mesh = plsc.VectorSubcoreMesh(core_axis_name="c", subcore_axis_name="s")


@functools.partial(
    pl.kernel,
    out_type=jax.ShapeDtypeStruct((NC, N_PAD), jnp.float32),
    mesh=_mesh,
    scratch_types=[
        pltpu.VMEM((2, E_MAIN), jnp.int32),
        pltpu.VMEM((E_MAIN,), jnp.int32),
        pltpu.VMEM((2, 128), jnp.int32),
        pltpu.VMEM((128,), jnp.int32),
        pltpu.VMEM((E_MAIN,), jnp.float32),
        pltpu.VMEM((Z_W,), jnp.float32),
        pltpu.VMEM_SHARED((N_PAD,), jnp.float32),
        pltpu.SemaphoreType.DMA,
        pltpu.SemaphoreType.DMA,
    ],
)
def _degree_kernel(edge_hbm, out_hbm, idx_v, idx1_v, tail_v, tail1_v,
                   ones_v, zeros_v, hist_sh, sem_idx, sem_sc):
    cid = lax.axis_index("c")
    sid = lax.axis_index("s")
    wid = sid * NC + cid

    h_idx = pltpu.async_copy(
        edge_hbm.at[pl.ds(0, 2), pl.ds(wid * E_MAIN, E_MAIN)],
        idx_v, sem_idx)

    @pl.when(wid < TAIL_W)
    def _():
        pltpu.sync_copy(
            edge_hbm.at[pl.ds(0, 2), pl.ds(NW * E_MAIN + wid * 128, 128)],
            tail_v)

    def fill(i, _):
        ones_v[pl.ds(i * 16, 16)] = jnp.ones((16,), jnp.float32)
        return 0

    lax.fori_loop(0, E_MAIN // 16, fill, 0)

    def fillz(i, _):
        zeros_v[pl.ds(i * 16, 16)] = jnp.zeros((16,), jnp.float32)
        return 0

    lax.fori_loop(0, Z_W // 16, fillz, 0)

    pltpu.sync_copy(zeros_v, hist_sh.at[pl.ds(sid * Z_W, Z_W)])
    h_idx.wait()
    plsc.subcore_barrier()

    @pl.when(wid < TAIL_W)
    def _():
        def deint_t(k, _):
            tail1_v[pl.ds(k * 16, 16)] = tail_v[1, pl.ds(k * 16, 16)]
            return 0

        lax.fori_loop(0, 128 // 16, deint_t, 0)
        pltpu.sync_copy(ones_v.at[pl.ds(0, 128)],
                        hist_sh.at[tail1_v], add=True)

    fires = []
    for c in range(N_CH):
        lo = c * CH

        def deint(k, _, lo=lo):
            idx1_v[pl.ds(lo + k * 16, 16)] = idx_v[1, pl.ds(lo + k * 16, 16)]
            return 0

        lax.fori_loop(0, CH // 16, deint, 0)
        fires.append(pltpu.async_copy(ones_v.at[pl.ds(lo, CH)],
                                      hist_sh.at[idx1_v.at[pl.ds(lo, CH)]],
                                      sem_sc, add=True))
    for h in fires:
        h.wait()

    plsc.subcore_barrier()

    @pl.when(sid == 0)
    def _():
        pltpu.sync_copy(hist_sh, out_hbm.at[cid])


def _mm_body(x_ref, w_ref, o_ref):
    o_ref[...] = jnp.dot(x_ref[...], w_ref[...],
                         preferred_element_type=jnp.float32)


def _scale_body(c_ref, xw_ref, o_ref):
    cnt = c_ref[0, :] + c_ref[1, :]
    o_ref[...] = xw_ref[...] * cnt[:, None]


def kernel(edge_index, x, W):
    deg = _degree_kernel(edge_index)

    xw = pl.pallas_call(
        _mm_body,
        grid=(N_BLKS,),
        out_shape=jax.ShapeDtypeStruct((N_NODES, D_FEAT), jnp.float32),
        in_specs=[
            pl.BlockSpec((ROWS_BLK, D_FEAT), lambda i: (i, 0)),
            pl.BlockSpec((D_FEAT, D_FEAT), lambda i: (0, 0)),
        ],
        out_specs=pl.BlockSpec((ROWS_BLK, D_FEAT), lambda i: (i, 0)),
    )(x, W)

    out = pl.pallas_call(
        _scale_body,
        grid=(N_BLKS,),
        out_shape=jax.ShapeDtypeStruct((N_NODES, D_FEAT), jnp.float32),
        in_specs=[
            pl.BlockSpec((NC, ROWS_BLK), lambda i: (0, i)),
            pl.BlockSpec((ROWS_BLK, D_FEAT), lambda i: (i, 0)),
        ],
        out_specs=pl.BlockSpec((ROWS_BLK, D_FEAT), lambda i: (i, 0)),
    )(deg, xw)
    return out
